# baseline (speedup 1.0000x reference)
import jax
import jax.numpy as jnp
from jax import lax
from jax.experimental import pallas as pl
from jax.experimental.pallas import tpu as pltpu
from jax.experimental.pallas import tpu_sc as plsc

_B = 4096
_D = 784
_DP = 896
_H = 512
_O = 10
_E = 8
_T = 256
_NT = 23
_NTP = 32
_PMAX = _NT * _T
_CB = 512
_OP = 128
_EPS = 2.220446049250313e-16

_NC = 2
_NS = 16
_NW = _NC * _NS
_BPW = _B // _NW


def _router_body(x_ref, wg_ref, pos_ref, xg_ref, eot_ref, act_ref):
    xx = x_ref[...]
    wg = wg_ref[...]
    logits = jnp.dot(xx, wg, preferred_element_type=jnp.float32)
    m = jnp.max(logits, axis=1, keepdims=True)
    el = jnp.exp(logits - m)
    probs = el / jnp.sum(el, axis=1, keepdims=True)
    pmax = jnp.max(probs, axis=1, keepdims=True)

    eids = lax.broadcasted_iota(jnp.int32, (_B, _E), 1)
    is_max = probs == pmax
    arg = jnp.min(jnp.where(is_max, eids, _E), axis=1, keepdims=True)
    oh = (eids == arg).astype(jnp.float32)

    tri = (lax.broadcasted_iota(jnp.int32, (_CB, _CB), 1)
           <= lax.broadcasted_iota(jnp.int32, (_CB, _CB), 0)).astype(jnp.float32)
    base = jnp.zeros((1, _E), dtype=jnp.float32)
    rank_parts = []
    for k in range(_B // _CB):
        blk = oh[k * _CB:(k + 1) * _CB, :]
        intra = jnp.dot(tri, blk, preferred_element_type=jnp.float32)
        cumk = intra + base
        rank_parts.append(jnp.sum(cumk * blk, axis=1, keepdims=True) - 1.0)
        base = cumk[_CB - 1:_CB, :]
    rank = jnp.concatenate(rank_parts, axis=0)
    counts_i = base.astype(jnp.int32)

    aligned_i = ((counts_i + (_T - 1)) >> 8) << 8
    excl = (lax.broadcasted_iota(jnp.int32, (_E, _E), 0)
            < lax.broadcasted_iota(jnp.int32, (_E, _E), 1)).astype(jnp.float32)
    po_f = jnp.dot(aligned_i.astype(jnp.float32), excl,
                   preferred_element_type=jnp.float32)
    po_i = po_f.astype(jnp.int32)

    po_tok = jnp.sum(oh * po_f, axis=1, keepdims=True)
    pos_ref[...] = (po_tok + rank).astype(jnp.int32)
    gate = pmax / (pmax + 1e-6)
    xg_ref[...] = jnp.concatenate(
        [xx, gate, jnp.zeros((_B, _DP - _D - 1), jnp.float32)], axis=1)

    s = lax.broadcasted_iota(jnp.int32, (_NTP, 1), 0) * _T
    e_t = jnp.sum((po_i <= s).astype(jnp.int32), axis=1, keepdims=True) - 1
    end_i = po_i + counts_i
    oh_t = (lax.broadcasted_iota(jnp.int32, (_NTP, _E), 1) == e_t)
    end_t = jnp.sum(jnp.where(oh_t, jnp.broadcast_to(end_i, (_NTP, _E)), 0),
                    axis=1, keepdims=True)
    eot_ref[...] = e_t
    act_ref[...] = (s < end_t).astype(jnp.int32)


def _router(x, w_gate):
    return pl.pallas_call(
        _router_body,
        out_shape=(
            jax.ShapeDtypeStruct((_B, 1), jnp.int32),
            jax.ShapeDtypeStruct((_B, _DP), jnp.float32),
            jax.ShapeDtypeStruct((_NTP, 1), jnp.int32),
            jax.ShapeDtypeStruct((_NTP, 1), jnp.int32),
        ),
    )(x, w_gate)


def _dispatch_body(x_hbm, pos_hbm, out_hbm, idx_v, rows_v, sem):
    wid = lax.axis_index("s") * _NC + lax.axis_index("c")
    base = wid * _BPW
    pltpu.sync_copy(pos_hbm.at[pl.ds(base, _BPW)], idx_v)
    pltpu.sync_copy(x_hbm.at[pl.ds(base, _BPW)], rows_v)
    pltpu.async_copy(rows_v, out_hbm.at[idx_v], sem).wait()


def _dispatch(xg, pos):
    mesh = plsc.VectorSubcoreMesh(core_axis_name="c", subcore_axis_name="s")
    return pl.kernel(
        _dispatch_body,
        out_type=jax.ShapeDtypeStruct((_PMAX, _DP), jnp.float32),
        mesh=mesh,
        scratch_types=[
            pltpu.VMEM((_BPW,), jnp.int32),
            pltpu.VMEM((_BPW, _DP), jnp.float32),
            pltpu.SemaphoreType.DMA,
        ],
    )(xg, pos)


def _mlp_body(eot_ref, act_ref, x_ref, w1_ref, b1_ref, w2_ref, b2_ref,
              w3_ref, b3_ref, out_ref):
    t = pl.program_id(0)

    @pl.when(act_ref[t] != 0)
    def _():
        bf = jnp.bfloat16
        xt = x_ref[...]
        g = xt[:, _D:_D + 1]
        w1p = jnp.concatenate(
            [w1_ref[0], jnp.zeros((_DP - _D, _H), jnp.float32)], axis=0)
        w3p = jnp.concatenate(
            [w3_ref[0], jnp.zeros((_H, _OP - _O), jnp.float32)], axis=1)
        b3p = jnp.concatenate(
            [b3_ref[0], jnp.zeros((1, _OP - _O), jnp.float32)], axis=1)
        h1 = jnp.maximum(
            jnp.dot(xt.astype(bf), w1p.astype(bf),
                    preferred_element_type=jnp.float32)
            + b1_ref[0], 0.0)
        h2 = jnp.maximum(
            jnp.dot(h1.astype(bf), w2_ref[0].astype(bf),
                    preferred_element_type=jnp.float32)
            + b2_ref[0], 0.0)
        o = (jnp.dot(h2.astype(bf), w3p.astype(bf),
                     preferred_element_type=jnp.float32)
             + b3p)
        v = jnp.exp(o) * g
        out_ref[...] = jnp.log(jnp.where(v == 0.0, jnp.float32(_EPS), v))


def _grouped_mlp(xs, W1, b1r, W2, b2r, W3, b3r, eot, act):
    grid_spec = pltpu.PrefetchScalarGridSpec(
        num_scalar_prefetch=2,
        grid=(_NT,),
        in_specs=[
            pl.BlockSpec((_T, _DP), lambda t, eot, act: (t, 0)),
            pl.BlockSpec((1, _D, _H), lambda t, eot, act: (eot[t], 0, 0)),
            pl.BlockSpec((1, 1, _H), lambda t, eot, act: (eot[t], 0, 0)),
            pl.BlockSpec((1, _H, _H), lambda t, eot, act: (eot[t], 0, 0)),
            pl.BlockSpec((1, 1, _H), lambda t, eot, act: (eot[t], 0, 0)),
            pl.BlockSpec((1, _H, _O), lambda t, eot, act: (eot[t], 0, 0)),
            pl.BlockSpec((1, 1, _O), lambda t, eot, act: (eot[t], 0, 0)),
        ],
        out_specs=pl.BlockSpec((_T, _OP), lambda t, eot, act: (t, 0)),
    )
    return pl.pallas_call(
        _mlp_body,
        grid_spec=grid_spec,
        out_shape=jax.ShapeDtypeStruct((_PMAX, _OP), jnp.float32),
        compiler_params=pltpu.CompilerParams(
            dimension_semantics=("arbitrary",)),
    )(eot, act, xs, W1, b1r, W2, b2r, W3, b3r)


def _combine_body(eo_hbm, pos_hbm, out_hbm, idx_v, rows_v, sem):
    wid = lax.axis_index("s") * _NC + lax.axis_index("c")
    base = wid * _BPW
    pltpu.sync_copy(pos_hbm.at[pl.ds(base, _BPW)], idx_v)
    pltpu.async_copy(eo_hbm.at[idx_v], rows_v, sem).wait()
    pltpu.sync_copy(rows_v, out_hbm.at[pl.ds(base, _BPW)])


def _combine(eo, pos):
    mesh = plsc.VectorSubcoreMesh(core_axis_name="c", subcore_axis_name="s")
    return pl.kernel(
        _combine_body,
        out_type=jax.ShapeDtypeStruct((_B, _OP), jnp.float32),
        mesh=mesh,
        scratch_types=[
            pltpu.VMEM((_BPW,), jnp.int32),
            pltpu.VMEM((_BPW, _OP), jnp.float32),
            pltpu.SemaphoreType.DMA,
        ],
    )(eo, pos)


def kernel(x, w_gate, W1, b1, W2, b2, W3, b3):
    pos2, xg, eot2, act2 = _router(x, w_gate)
    pos = pos2.reshape(_B)
    xs = _dispatch(xg, pos)
    eo = _grouped_mlp(xs, W1, b1.reshape(_E, 1, _H), W2,
                      b2.reshape(_E, 1, _H), W3, b3.reshape(_E, 1, _O),
                      eot2.reshape(_NTP), act2.reshape(_NTP))
    out_tok = _combine(eo, pos)
    return out_tok[:, :_O]


# speedup vs baseline: 1.2337x; 1.2337x over previous
import jax
import jax.numpy as jnp
from jax import lax
from jax.experimental import pallas as pl
from jax.experimental.pallas import tpu as pltpu
from jax.experimental.pallas import tpu_sc as plsc

_B = 4096
_D = 784
_DP = 1024
_DPW = _DP // 2
_H = 512
_O = 10
_E = 8
_T = 256
_NT = 23
_NTP = 32
_PMAX = _NT * _T
_CB = 512
_OP = 128
_EPS = 2.220446049250313e-16

_NC = 2
_NS = 16
_NW = _NC * _NS
_BPW = _B // _NW


_CH = 256
_NCH = _B // _CH


def _router_body(xt_ref, wg_ref, xg_ref, pos_ref, eot_ref, act_ref,
                 base_ref, rank_s, arg_s, po_s):
    t = pl.program_id(0)

    @pl.when(t == 0)
    def _():
        base_ref[...] = jnp.zeros((_E, 1), jnp.float32)

    @pl.when(t < _NCH)
    def _():
        xt_t = xt_ref[...]
        logits = lax.dot_general(
            wg_ref[...], xt_t, (((0,), (0,)), ((), ())),
            preferred_element_type=jnp.float32)
        m = jnp.max(logits, axis=0, keepdims=True)
        el = jnp.exp(logits - m)
        probs = el / jnp.sum(el, axis=0, keepdims=True)
        pmax = jnp.max(probs, axis=0, keepdims=True)
        eids = lax.broadcasted_iota(jnp.int32, (_E, _CH), 0)
        is_max = probs == pmax
        arg = jnp.min(jnp.where(is_max, eids, _E), axis=0, keepdims=True)
        oh = (eids == arg).astype(jnp.float32)

        tri = (lax.broadcasted_iota(jnp.int32, (_CH, _CH), 0)
               <= lax.broadcasted_iota(jnp.int32, (_CH, _CH), 1)
               ).astype(jnp.float32)
        intra = jnp.dot(oh, tri, preferred_element_type=jnp.float32)
        cumk = intra + base_ref[...]
        base_ref[...] = cumk[:, _CH - 1:_CH]
        rank = jnp.sum(cumk * oh, axis=0, keepdims=True) - 1.0
        rank_s[pl.ds(t, 1), :] = rank
        arg_s[pl.ds(t, 1), :] = arg

        def rb(u):
            return (u + 0x7FFF + ((u >> 16) & 1)) & jnp.int32(-65536)

        hw = _D // 2
        u_all = lax.bitcast_convert_type(xt_t, jnp.int32)
        lo_w = lax.shift_right_logical(rb(u_all[0:hw, :]), 16)
        hi_w = rb(u_all[hw:_D, :])
        words = hi_w | lo_w

        gate = pmax / (pmax + 1e-6)
        ghb = rb(lax.bitcast_convert_type(gate, jnp.int32))
        g_hi_val = lax.bitcast_convert_type(ghb, jnp.float32)
        glb = rb(lax.bitcast_convert_type(gate - g_hi_val, jnp.int32))
        gate_word = glb | lax.shift_right_logical(ghb, 16)

        xw = jnp.concatenate(
            [words, gate_word,
             jnp.zeros((_DPW - hw - 1, _CH), jnp.int32)], axis=0)
        xg_ref[...] = jnp.transpose(xw)

    @pl.when(t == _NCH - 1)
    def _():
        counts_i = jnp.transpose(base_ref[...]).astype(jnp.int32)
        aligned_i = ((counts_i + (_T - 1)) >> 8) << 8
        excl = (lax.broadcasted_iota(jnp.int32, (_E, _E), 0)
                < lax.broadcasted_iota(jnp.int32, (_E, _E), 1)
                ).astype(jnp.float32)
        po_f = jnp.dot(aligned_i.astype(jnp.float32), excl,
                       preferred_element_type=jnp.float32)
        po_s[...] = po_f
        po_i = po_f.astype(jnp.int32)
        s = lax.broadcasted_iota(jnp.int32, (_NTP, 1), 0) * _T
        e_t = jnp.sum((po_i <= s).astype(jnp.int32), axis=1, keepdims=True) - 1
        end_i = po_i + counts_i
        oh_t = (lax.broadcasted_iota(jnp.int32, (_NTP, _E), 1) == e_t)
        end_t = jnp.sum(jnp.where(oh_t, jnp.broadcast_to(end_i, (_NTP, _E)), 0),
                        axis=1, keepdims=True)
        eot_ref[...] = e_t
        act_ref[...] = (s < end_t).astype(jnp.int32)

    @pl.when(t == _NCH)
    def _():
        arg_all = arg_s[...]
        po_v = po_s[...]
        acc = jnp.zeros((_NCH, _CH), jnp.float32)
        for e in range(_E):
            acc = acc + jnp.where(arg_all == e, po_v[0:1, e:e + 1], 0.0)
        pos_ref[...] = (acc + rank_s[...]).astype(jnp.int32)


def _router(xt, w_gate):
    return pl.pallas_call(
        _router_body,
        grid=(_NCH + 1,),
        in_specs=[
            pl.BlockSpec((_D, _CH), lambda t: (0, jnp.minimum(t, _NCH - 1))),
            pl.BlockSpec((_D, _E), lambda t: (0, 0)),
        ],
        out_specs=(
            pl.BlockSpec((_CH, _DPW), lambda t: (jnp.minimum(t, _NCH - 1), 0)),
            pl.BlockSpec((_NCH, _CH), lambda t: (0, 0)),
            pl.BlockSpec((_NTP, 1), lambda t: (0, 0)),
            pl.BlockSpec((_NTP, 1), lambda t: (0, 0)),
        ),
        out_shape=(
            jax.ShapeDtypeStruct((_B, _DPW), jnp.int32),
            jax.ShapeDtypeStruct((_NCH, _CH), jnp.int32),
            jax.ShapeDtypeStruct((_NTP, 1), jnp.int32),
            jax.ShapeDtypeStruct((_NTP, 1), jnp.int32),
        ),
        scratch_shapes=[
            pltpu.VMEM((_E, 1), jnp.float32),
            pltpu.VMEM((_NCH, _CH), jnp.float32),
            pltpu.VMEM((_NCH, _CH), jnp.int32),
            pltpu.VMEM((1, _E), jnp.float32),
        ],
        compiler_params=pltpu.CompilerParams(
            dimension_semantics=("arbitrary",)),
    )(xt, w_gate)


def _dispatch_body(x_hbm, pos_hbm, out_hbm, idx_v, rows_v, sem):
    wid = lax.axis_index("s") * _NC + lax.axis_index("c")
    base = wid * _BPW
    pltpu.sync_copy(pos_hbm.at[pl.ds(base, _BPW)], idx_v)
    pltpu.sync_copy(x_hbm.at[pl.ds(base, _BPW)], rows_v)
    pltpu.async_copy(rows_v, out_hbm.at[idx_v], sem).wait()


def _dispatch(xg, pos):
    mesh = plsc.VectorSubcoreMesh(core_axis_name="c", subcore_axis_name="s")
    return pl.kernel(
        _dispatch_body,
        out_type=jax.ShapeDtypeStruct((_PMAX, _DPW), jnp.int32),
        mesh=mesh,
        scratch_types=[
            pltpu.VMEM((_BPW,), jnp.int32),
            pltpu.VMEM((_BPW, _DPW), jnp.int32),
            pltpu.SemaphoreType.DMA,
        ],
    )(xg, pos)


def _mlp_body(eot_ref, act_ref, x_ref, w1_ref, b1_ref, w2_ref, b2_ref,
              w3_ref, b3_ref, out_ref):
    t = pl.program_id(0)

    @pl.when(act_ref[t] != 0)
    def _():
        e = eot_ref[t]
        hw = _D // 2
        xw = x_ref[...]
        f_lo = lax.bitcast_convert_type(xw << 16, jnp.float32)
        f_hi = lax.bitcast_convert_type(
            xw & jnp.int32(-65536), jnp.float32)
        g = f_lo[:, hw:hw + 1] + f_hi[:, hw:hw + 1]
        zpad = jnp.zeros((_DPW - hw, _H), jnp.float32)
        wev = jnp.concatenate([w1_ref[e][0:hw, :], zpad], axis=0)
        wod = jnp.concatenate([w1_ref[e][hw:_D, :], zpad], axis=0)
        w3p = jnp.concatenate(
            [w3_ref[e], jnp.zeros((_H, _OP - _O), jnp.float32)], axis=1)
        b3p = jnp.concatenate(
            [b3_ref[e], jnp.zeros((1, _OP - _O), jnp.float32)], axis=1)
        h1 = jnp.maximum(
            jnp.dot(f_lo, wev, preferred_element_type=jnp.float32)
            + jnp.dot(f_hi, wod, preferred_element_type=jnp.float32)
            + b1_ref[e], 0.0)
        h2 = jnp.maximum(
            jnp.dot(h1, w2_ref[e], preferred_element_type=jnp.float32)
            + b2_ref[e], 0.0)
        o = (jnp.dot(h2, w3p, preferred_element_type=jnp.float32)
             + b3p)
        v = jnp.exp(o) * g
        out_ref[...] = jnp.log(jnp.where(v == 0.0, jnp.float32(_EPS), v))


def _grouped_mlp(xs, W1, b1r, W2, b2r, W3, b3r, eot, act):
    grid_spec = pltpu.PrefetchScalarGridSpec(
        num_scalar_prefetch=2,
        grid=(_NT,),
        in_specs=[
            pl.BlockSpec((_T, _DPW), lambda t, eot, act: (t, 0)),
            pl.BlockSpec((_E, _D, _H), lambda t, eot, act: (0, 0, 0)),
            pl.BlockSpec((_E, 1, _H), lambda t, eot, act: (0, 0, 0)),
            pl.BlockSpec((_E, _H, _H), lambda t, eot, act: (0, 0, 0)),
            pl.BlockSpec((_E, 1, _H), lambda t, eot, act: (0, 0, 0)),
            pl.BlockSpec((_E, _H, _O), lambda t, eot, act: (0, 0, 0)),
            pl.BlockSpec((_E, 1, _O), lambda t, eot, act: (0, 0, 0)),
        ],
        out_specs=pl.BlockSpec((_T, _OP), lambda t, eot, act: (t, 0)),
    )
    return pl.pallas_call(
        _mlp_body,
        grid_spec=grid_spec,
        out_shape=jax.ShapeDtypeStruct((_PMAX, _OP), jnp.float32),
        compiler_params=pltpu.CompilerParams(
            dimension_semantics=("arbitrary",)),
    )(eot, act, xs, W1, b1r, W2, b2r, W3, b3r)


def _combine_body(eo_hbm, pos_hbm, out_hbm, idx_v, rows_v, sem):
    wid = lax.axis_index("s") * _NC + lax.axis_index("c")
    base = wid * _BPW
    pltpu.sync_copy(pos_hbm.at[pl.ds(base, _BPW)], idx_v)
    pltpu.async_copy(eo_hbm.at[idx_v], rows_v, sem).wait()
    pltpu.sync_copy(rows_v, out_hbm.at[pl.ds(base, _BPW)])


def _combine(eo, pos):
    mesh = plsc.VectorSubcoreMesh(core_axis_name="c", subcore_axis_name="s")
    return pl.kernel(
        _combine_body,
        out_type=jax.ShapeDtypeStruct((_B, _OP), jnp.float32),
        mesh=mesh,
        scratch_types=[
            pltpu.VMEM((_BPW,), jnp.int32),
            pltpu.VMEM((_BPW, _OP), jnp.float32),
            pltpu.SemaphoreType.DMA,
        ],
    )(eo, pos)


def kernel(x, w_gate, W1, b1, W2, b2, W3, b3):
    xg, pos2, eot2, act2 = _router(x.T, w_gate)
    pos = pos2.reshape(_B)
    xs = _dispatch(xg, pos)
    eo = _grouped_mlp(xs, W1,
                      b1.reshape(_E, 1, _H), W2,
                      b2.reshape(_E, 1, _H), W3, b3.reshape(_E, 1, _O),
                      eot2.reshape(_NTP), act2.reshape(_NTP))
    out_tok = _combine(eo, pos)
    return out_tok[:, :_O]
